# baseline (speedup 1.0000x reference)
import functools
import jax
import jax.numpy as jnp
from jax import lax
from jax.experimental import pallas as pl
from jax.experimental.pallas import tpu as pltpu
from jax.experimental.pallas import tpu_sc as plsc

PAD = 1
L = 16
CH = 32
BLK = 512
FAST_CORE = 0
NCH_F = 9
NCH_S = BLK // CH - NCH_F
TOKW = 384


def _make_sc(bsz, seq_len, d):
    info = plsc.get_sparse_core_info()
    nc = info.num_cores
    ns = info.num_subcores
    assert nc == 2 and ns * BLK == seq_len
    mesh = plsc.VectorSubcoreMesh(core_axis_name="c", subcore_axis_name="s")

    @functools.partial(
        pl.kernel,
        mesh=mesh,
        out_type=jax.ShapeDtypeStruct((bsz * seq_len, d), jnp.float32),
        scratch_types=[
            pltpu.VMEM((bsz, TOKW), jnp.int32),
            pltpu.VMEM((CH, d), jnp.float32),
            pltpu.VMEM((CH, d), jnp.float32),
            pltpu.VMEM((NCH_F, CH), jnp.int32),
            pltpu.VMEM((CH,), jnp.int32),
            pltpu.SemaphoreType.DMA,
            pltpu.SemaphoreType.DMA,
            pltpu.SemaphoreType.DMA,
        ],
    )
    def k(inp_hbm, table_hbm, out_hbm, tok_v, buf0, buf1, iidx, fidx,
          g0, g1, ssem):
        cid = lax.axis_index("c")
        sid = lax.axis_index("s")
        is_fast = cid == FAST_CORE
        blk0 = pl.multiple_of(sid * BLK, BLK)
        base = blk0 + jnp.where(is_fast, 0, NCH_F * CH)
        base = pl.multiple_of(base, CH)
        twin = blk0 + jnp.where(is_fast, 0, 256)
        tshift = jnp.where(is_fast, 0, NCH_F * CH - 256)
        my_nch = jnp.where(is_fast, NCH_F, NCH_S)

        @pl.when(is_fast)
        def _stage_f():
            for b in range(bsz):
                pltpu.sync_copy(
                    inp_hbm.at[pl.ds(b * seq_len + twin, TOKW)], tok_v.at[b]
                )

        @pl.when(jnp.logical_not(is_fast))
        def _stage_s():
            for b in range(bsz):
                pltpu.sync_copy(
                    inp_hbm.at[pl.ds(b * seq_len + twin, 256)],
                    tok_v.at[b].at[pl.ds(0, 256)],
                )

        bufs = (buf0, buf1)
        gsems = (g0, g1)
        lane = jnp.arange(L, dtype=jnp.int32)

        for ch in range(NCH_F):
            for v in range(CH // L):
                iidx[ch, pl.ds(v * L, L)] = lane + (base + ch * CH + v * L + 2)

        lo = tshift
        hi = tshift + my_nch * CH
        has_pad = []
        for b in range(bsz):
            acc = jnp.zeros((L,), jnp.int32)
            for v in range(TOKW // L):
                tok = tok_v[b, pl.ds(v * L, L)]
                col = lane + v * L
                inrange = (col >= lo) & (col < hi)
                acc = acc | jnp.where(inrange & (tok == PAD), 1, 0)
            s = acc[0]
            for i in range(1, L):
                s = s | acc[i]
            has_pad.append(s > 0)

        def clean_gather(ch, p):
            return pltpu.make_async_copy(
                table_hbm.at[iidx.at[ch]], bufs[p], gsems[p]
            )

        def out_slice(b, ch):
            start = pl.multiple_of(b * seq_len + base + ch * CH, CH)
            return out_hbm.at[pl.ds(start, CH)]

        def chunk_body(ch, prefetch_next):
            p = ch % 2
            clean_gather(ch, p).wait()
            if prefetch_next:
                clean_gather(ch + 1, 1 - p).start()
            for b in range(bsz):
                pltpu.make_async_copy(bufs[p], out_slice(b, ch), ssem).start()
            for b in range(bsz):
                pltpu.make_async_copy(bufs[p], out_slice(b, ch), ssem).wait()

        clean_gather(0, 0).start()
        for ch in range(NCH_S):
            chunk_body(ch, prefetch_next=(ch + 1 < NCH_S))

        @pl.when(is_fast)
        def _tail():
            clean_gather(NCH_S, NCH_S % 2).start()
            for ch in range(NCH_S, NCH_F):
                chunk_body(ch, prefetch_next=(ch + 1 < NCH_F))

        def fixup_chunk(b, ch, tok_off):
            for v in range(CH // L):
                tok = tok_v[b, pl.ds(tok_off + ch * CH + v * L, L)]
                pos = lane + (base + ch * CH + v * L + 2)
                fidx[pl.ds(v * L, L)] = jnp.where(tok != PAD, pos, PAD)
            pltpu.make_async_copy(table_hbm.at[fidx], buf0, g0).start()
            pltpu.make_async_copy(table_hbm.at[fidx], buf0, g0).wait()
            pltpu.make_async_copy(buf0, out_slice(b, ch), ssem).start()
            pltpu.make_async_copy(buf0, out_slice(b, ch), ssem).wait()

        for b in range(bsz):

            @pl.when(has_pad[b] & is_fast)
            def _fixup_f(b=b):
                for ch in range(NCH_F):
                    fixup_chunk(b, ch, 0)

            @pl.when(has_pad[b] & jnp.logical_not(is_fast))
            def _fixup_s(b=b):
                for ch in range(NCH_S):
                    fixup_chunk(b, ch, NCH_F * CH - 256)

    return k


def kernel(input, weights):
    bsz, seq_len = input.shape
    d = weights.shape[1]
    k = _make_sc(bsz, seq_len, d)
    out = k(input.reshape(-1), weights)
    return out.reshape(bsz, seq_len, d)
